# baseline (speedup 1.0000x reference)
import functools
import math

import jax
import jax.numpy as jnp
from jax import lax
from jax.experimental import pallas as pl
from jax.experimental.pallas import tpu as pltpu
from jax.experimental.pallas import tpu_sc as plsc

_BTA = 256
_BT = 256
_NHC = 2


def _gelu_exact(h):
    return 0.5 * h * (1.0 + lax.erf(h * (1.0 / math.sqrt(2.0))))


def _gate_body(x_ref, wgr_ref, wg_ref, pos_ref, ss_ref, meta_ref,
               idx_s, rank_s, sc_s, cnt_s, *, n_experts, n_tiles):
    ph = pl.program_id(0)
    i = pl.program_id(1)
    rows = pl.ds(i * _BTA, _BTA)

    @pl.when(ph == 0)
    def _phase0():
        @pl.when(i == 0)
        def _init():
            cnt_s[...] = jnp.zeros((1, 16), jnp.float32)

        xb = x_ref[...]
        g = lax.dot_general(
            xb, wgr_ref[...], (((1,), (1,)), ((), ())),
            preferred_element_type=jnp.float32)
        wgn = wg_ref[...]
        norm = jnp.sqrt(jnp.sum(wgn * wgn, axis=1, keepdims=True))
        wgn = wgn / jnp.maximum(norm, 1e-4)
        logits = lax.dot_general(
            g, wgn, (((1,), (1,)), ((), ())),
            preferred_element_type=jnp.float32)
        m = jnp.max(logits, axis=1, keepdims=True)
        s = jnp.sum(jnp.exp(logits - m), axis=1, keepdims=True)
        lane_e = lax.broadcasted_iota(jnp.int32, logits.shape, 1)
        idx = jnp.min(jnp.where(logits == m, lane_e, n_experts),
                      axis=1, keepdims=True)
        idx_s[rows, :] = idx
        sc_s[rows, :] = 1.0 / s

        lane16 = lax.broadcasted_iota(jnp.int32, (_BTA, 16), 1)
        onehot = (lane16 == idx).astype(jnp.float32)
        r_io = lax.broadcasted_iota(jnp.int32, (_BTA, _BTA), 0)
        c_io = lax.broadcasted_iota(jnp.int32, (_BTA, _BTA), 1)
        tri = (r_io > c_io).astype(jnp.bfloat16)
        ranks = lax.dot_general(
            tri, onehot.astype(jnp.bfloat16), (((1,), (0,)), ((), ())),
            preferred_element_type=jnp.float32)
        ranks = ranks + cnt_s[...]
        rank_s[rows, :] = jnp.sum(ranks * onehot, axis=1, keepdims=True)
        cnt_s[...] += jnp.sum(onehot, axis=0, keepdims=True)

    @pl.when(ph == 1)
    def _phase1():
        cnt = cnt_s[...]
        a_io = lax.broadcasted_iota(jnp.int32, (16, 16), 0)
        b_io = lax.broadcasted_iota(jnp.int32, (16, 16), 1)
        hp = lax.Precision.HIGHEST
        up = (a_io < b_io).astype(jnp.float32)
        eye = (a_io == b_io).astype(jnp.float32)
        offs = lax.dot_general(
            cnt, up, (((1,), (0,)), ((), ())), precision=hp,
            preferred_element_type=jnp.float32)

        bt_f = float(_BT)
        incl = offs + cnt
        jfirst = jnp.floor(offs / bt_f)
        jlast = jnp.floor((incl - 1.0) / bt_f)
        nonempty = incl > offs
        ntiles = jnp.where(nonempty, jlast - jfirst + 1.0, 0.0)
        cumt = lax.dot_general(
            ntiles, up, (((1,), (0,)), ((), ())), precision=hp,
            preferred_element_type=jnp.float32)

        def col(v):
            return lax.dot_general(eye, v, (((1,), (1,)), ((), ())),
                                   precision=hp,
                                   preferred_element_type=jnp.float32)

        s_io = b_io.astype(jnp.float32)
        ind = jnp.logical_and(s_io >= col(cumt),
                              s_io < col(cumt) + col(ntiles))
        ind = ind.astype(jnp.float32)

        def rowdot(v):
            return lax.dot_general(v, ind, (((1,), (0,)), ((), ())),
                                   precision=hp,
                                   preferred_element_type=jnp.float32)

        ev = lax.broadcasted_iota(jnp.int32, (1, 16), 1).astype(jnp.float32)
        ones = jnp.ones((1, 16), jnp.float32)
        valid = rowdot(ones)
        be = rowdot(ev)
        bj = rowdot(jfirst - cumt) + ev
        obe = rowdot(offs)
        onx = rowdot(incl)
        be = jnp.where(valid > 0, be, float(n_experts - 1))
        bj = jnp.where(valid > 0, bj, float(n_tiles - 1))
        blo = jnp.clip(obe - bj * bt_f, 0.0, bt_f)
        bhi = jnp.clip(onx - bj * bt_f, 0.0, bt_f)
        meta_ref[...] = jnp.concatenate(
            [be, bj, blo, bhi], axis=0).astype(jnp.int32)

        idx = idx_s[rows, :]
        lane16 = lax.broadcasted_iota(jnp.int32, (_BTA, 16), 1)
        onehot = (lane16 == idx).astype(jnp.float32)
        og = jnp.sum(onehot * offs, axis=1, keepdims=True)
        pos_ref[...] = (rank_s[rows, :] + og).astype(jnp.int32)
        ss_ref[...] = jnp.broadcast_to(sc_s[rows, :], (_BTA, 128))


def _gate_positions(x, wg_red, wg, n_experts):
    t, d = x.shape
    n_tiles = t // _BTA
    tile_map = lambda p, i: (jnp.where(p == 0, n_tiles, i), 0)
    pos_full, ss_full, meta = pl.pallas_call(
        functools.partial(_gate_body, n_experts=n_experts, n_tiles=n_tiles),
        grid=(2, n_tiles),
        in_specs=[
            pl.BlockSpec((_BTA, d), lambda p, i: (i, 0)),
            pl.BlockSpec(wg_red.shape, lambda p, i: (0, 0)),
            pl.BlockSpec(wg.shape, lambda p, i: (0, 0)),
        ],
        out_specs=[
            pl.BlockSpec((_BTA, 1), tile_map),
            pl.BlockSpec((_BTA, 128), tile_map),
            pl.BlockSpec((4, 16), lambda p, i: (0, 0)),
        ],
        out_shape=[
            jax.ShapeDtypeStruct((t + _BTA, 1), jnp.int32),
            jax.ShapeDtypeStruct((t + _BTA, 128), jnp.float32),
            jax.ShapeDtypeStruct((4, 16), jnp.int32),
        ],
        scratch_shapes=[
            pltpu.VMEM((t, 1), jnp.int32),
            pltpu.VMEM((t, 1), jnp.float32),
            pltpu.VMEM((t, 1), jnp.float32),
            pltpu.VMEM((1, 16), jnp.float32),
        ],
        compiler_params=pltpu.CompilerParams(
            dimension_semantics=("arbitrary", "arbitrary"),
        ),
    )(x, wg_red, wg)
    return pos_full[:t], ss_full[:t], meta


def _sc_scatter(x, ss16, pos):
    t, d = x.shape
    mesh = plsc.VectorSubcoreMesh(core_axis_name="c", subcore_axis_name="s")
    nw = 32
    b = t // nw

    @functools.partial(
        pl.kernel, mesh=mesh,
        out_type=[jax.ShapeDtypeStruct((t, d), jnp.float32),
                  jax.ShapeDtypeStruct((t, 128), jnp.float32)],
        scratch_types=[
            pltpu.VMEM((b,), jnp.int32),
            pltpu.VMEM((b, d), jnp.float32),
            pltpu.VMEM((b, 128), jnp.float32),
            pltpu.SemaphoreType.DMA,
        ],
    )
    def k(x_hbm, ss_hbm, pos_hbm, xs_hbm, sss_hbm, idx_v, rows_v, sc_v, sem):
        wid = lax.axis_index("s") * 2 + lax.axis_index("c")
        base = wid * b
        pltpu.sync_copy(pos_hbm.at[pl.ds(base, b)], idx_v)
        pltpu.sync_copy(x_hbm.at[pl.ds(base, b)], rows_v)
        pltpu.sync_copy(ss_hbm.at[pl.ds(base, b)], sc_v)
        pltpu.async_copy(rows_v, xs_hbm.at[idx_v], sem).wait()
        pltpu.async_copy(sc_v, sss_hbm.at[idx_v], sem).wait()

    return k(x, ss16, pos)


def _ffn_body(meta_ref, xs_ref, ss_ref, w1_ref, b1_ref, w2_ref,
              b2_ref, o_ref, *, n_experts):
    c = pl.program_id(0)
    s = pl.program_id(1)
    bj = meta_ref[16 + s]
    lo = meta_ref[32 + s]
    hi = meta_ref[48 + s]
    rows = pl.ds(bj * _BT, _BT)

    @pl.when(jnp.logical_and(c == 0, s == 0))
    def _zero():
        o_ref[...] = jnp.zeros_like(o_ref[...])

    @pl.when(hi > lo)
    def _compute():
        xb = xs_ref[rows, :].astype(jnp.bfloat16)
        w1 = w1_ref[0].astype(jnp.bfloat16)
        h = jnp.dot(xb, w1, preferred_element_type=jnp.float32)
        h = _gelu_exact(h + b1_ref[0])
        part = jnp.dot(h.astype(jnp.bfloat16), w2_ref[0].astype(jnp.bfloat16),
                       preferred_element_type=jnp.float32)
        part = part + jnp.where(c == 0, 1.0, 0.0) * b2_ref[0]
        y = part * ss_ref[rows, :1]
        r_io = lax.broadcasted_iota(jnp.int32, (_BT, 1), 0)
        mask = jnp.logical_and(r_io >= lo, r_io < hi)
        o_ref[rows, :] += jnp.where(mask, y, 0.0)


def _grouped_ffn(xs, sss, meta, weight1, bias1, weight2, bias2):
    t, d = xs.shape
    n_experts, _, hdim = weight1.shape
    hc = hdim // _NHC

    grid_spec = pltpu.PrefetchScalarGridSpec(
        num_scalar_prefetch=1,
        grid=(_NHC, 16),
        in_specs=[
            pl.BlockSpec((t, d), lambda c, s, m: (0, 0)),
            pl.BlockSpec((t, 128), lambda c, s, m: (0, 0)),
            pl.BlockSpec((1, d, hc), lambda c, s, m: (m[s], 0, c)),
            pl.BlockSpec((1, 1, hc), lambda c, s, m: (m[s], 0, c)),
            pl.BlockSpec((1, hc, d), lambda c, s, m: (m[s], c, 0)),
            pl.BlockSpec((1, 1, d), lambda c, s, m: (m[s], 0, 0)),
        ],
        out_specs=pl.BlockSpec((t, d), lambda c, s, m: (0, 0)),
    )
    body = functools.partial(_ffn_body, n_experts=n_experts)
    return pl.pallas_call(
        body,
        grid_spec=grid_spec,
        out_shape=jax.ShapeDtypeStruct((t, d), jnp.float32),
        compiler_params=pltpu.CompilerParams(
            dimension_semantics=("arbitrary", "arbitrary"),
        ),
    )(meta, xs, sss, weight1, bias1.reshape(n_experts, 1, hdim),
      weight2, bias2.reshape(n_experts, 1, d))


def _sc_gather(ys, pos):
    t, d = ys.shape
    mesh = plsc.VectorSubcoreMesh(core_axis_name="c", subcore_axis_name="s")
    nw = 32
    b = t // nw

    @functools.partial(
        pl.kernel, mesh=mesh,
        out_type=jax.ShapeDtypeStruct((t, d), jnp.float32),
        scratch_types=[
            pltpu.VMEM((b,), jnp.int32),
            pltpu.VMEM((b, d), jnp.float32),
            pltpu.SemaphoreType.DMA,
        ],
    )
    def k(ys_hbm, pos_hbm, y_hbm, idx_v, rows_v, sem):
        wid = lax.axis_index("s") * 2 + lax.axis_index("c")
        base = wid * b
        pltpu.sync_copy(pos_hbm.at[pl.ds(base, b)], idx_v)
        pltpu.async_copy(ys_hbm.at[idx_v], rows_v, sem).wait()
        pltpu.sync_copy(rows_v, y_hbm.at[pl.ds(base, b)])

    return k(ys, pos)


def kernel(hidden_states, wg_red, wg, weight1, bias1, weight2, bias2):
    bsz, t, d = hidden_states.shape
    n_experts = weight1.shape[0]
    x = hidden_states.reshape(t, d)

    pos, ss16, meta = _gate_positions(x, wg_red, wg, n_experts)
    pos1 = pos.reshape(t)
    xs, sss = _sc_scatter(x, ss16, pos1)
    ys = _grouped_ffn(xs, sss, meta.reshape(64),
                      weight1, bias1, weight2, bias2)
    y = _sc_gather(ys, pos1)
    return y.reshape(bsz, t, d)


# speedup vs baseline: 1.0697x; 1.0697x over previous
import functools
import math

import jax
import jax.numpy as jnp
from jax import lax
from jax.experimental import pallas as pl
from jax.experimental.pallas import tpu as pltpu
from jax.experimental.pallas import tpu_sc as plsc

_BTA = 512
_BT = 256
_NHC = 2


def _gelu_exact(h):
    return 0.5 * h * (1.0 + lax.erf(h * (1.0 / math.sqrt(2.0))))


def _gate_body(x_ref, wgr_ref, wg_ref, pos_ref, ss_ref, meta_ref,
               idx_s, rank_s, sc_s, cnt_s, *, n_experts, n_tiles):
    ph = pl.program_id(0)
    i = pl.program_id(1)
    rows = pl.ds(i * _BTA, _BTA)

    @pl.when(ph == 0)
    def _phase0():
        @pl.when(i == 0)
        def _init():
            cnt_s[...] = jnp.zeros((1, 16), jnp.float32)

        xb = x_ref[...]
        g = lax.dot_general(
            xb, wgr_ref[...], (((1,), (1,)), ((), ())),
            preferred_element_type=jnp.float32)
        wgn = wg_ref[...]
        norm = jnp.sqrt(jnp.sum(wgn * wgn, axis=1, keepdims=True))
        wgn = wgn / jnp.maximum(norm, 1e-4)
        logits = lax.dot_general(
            g, wgn, (((1,), (1,)), ((), ())),
            preferred_element_type=jnp.float32)
        m = jnp.max(logits, axis=1, keepdims=True)
        s = jnp.sum(jnp.exp(logits - m), axis=1, keepdims=True)
        lane_e = lax.broadcasted_iota(jnp.int32, logits.shape, 1)
        idx = jnp.min(jnp.where(logits == m, lane_e, n_experts),
                      axis=1, keepdims=True)
        idx_s[rows, :] = idx
        sc_s[rows, :] = 1.0 / s

        lane16 = lax.broadcasted_iota(jnp.int32, (_BTA, 16), 1)
        onehot = (lane16 == idx).astype(jnp.float32)
        r_io = lax.broadcasted_iota(jnp.int32, (_BTA, _BTA), 0)
        c_io = lax.broadcasted_iota(jnp.int32, (_BTA, _BTA), 1)
        tri = (r_io > c_io).astype(jnp.bfloat16)
        ranks = lax.dot_general(
            tri, onehot.astype(jnp.bfloat16), (((1,), (0,)), ((), ())),
            preferred_element_type=jnp.float32)
        ranks = ranks + cnt_s[...]
        rank_s[rows, :] = jnp.sum(ranks * onehot, axis=1, keepdims=True)
        cnt_s[...] += jnp.sum(onehot, axis=0, keepdims=True)

    @pl.when(ph == 1)
    def _phase1():
        cnt = cnt_s[...]
        a_io = lax.broadcasted_iota(jnp.int32, (16, 16), 0)
        b_io = lax.broadcasted_iota(jnp.int32, (16, 16), 1)
        hp = lax.Precision.HIGHEST
        up = (a_io < b_io).astype(jnp.float32)
        eye = (a_io == b_io).astype(jnp.float32)
        offs = lax.dot_general(
            cnt, up, (((1,), (0,)), ((), ())), precision=hp,
            preferred_element_type=jnp.float32)

        bt_f = float(_BT)
        incl = offs + cnt
        jfirst = jnp.floor(offs / bt_f)
        jlast = jnp.floor((incl - 1.0) / bt_f)
        nonempty = incl > offs
        ntiles = jnp.where(nonempty, jlast - jfirst + 1.0, 0.0)
        cumt = lax.dot_general(
            ntiles, up, (((1,), (0,)), ((), ())), precision=hp,
            preferred_element_type=jnp.float32)

        def col(v):
            return lax.dot_general(eye, v, (((1,), (1,)), ((), ())),
                                   precision=hp,
                                   preferred_element_type=jnp.float32)

        s_io = b_io.astype(jnp.float32)
        ind = jnp.logical_and(s_io >= col(cumt),
                              s_io < col(cumt) + col(ntiles))
        ind = ind.astype(jnp.float32)

        def rowdot(v):
            return lax.dot_general(v, ind, (((1,), (0,)), ((), ())),
                                   precision=hp,
                                   preferred_element_type=jnp.float32)

        ev = lax.broadcasted_iota(jnp.int32, (1, 16), 1).astype(jnp.float32)
        ones = jnp.ones((1, 16), jnp.float32)
        valid = rowdot(ones)
        be = rowdot(ev)
        bj = rowdot(jfirst - cumt) + ev
        obe = rowdot(offs)
        onx = rowdot(incl)
        be = jnp.where(valid > 0, be, float(n_experts - 1))
        bj = jnp.where(valid > 0, bj, float(n_tiles - 1))
        blo = jnp.clip(obe - bj * bt_f, 0.0, bt_f)
        bhi = jnp.clip(onx - bj * bt_f, 0.0, bt_f)
        meta_ref[...] = jnp.concatenate(
            [be, bj, blo, bhi], axis=0).astype(jnp.int32)

        idx = idx_s[rows, :]
        lane16 = lax.broadcasted_iota(jnp.int32, (_BTA, 16), 1)
        onehot = (lane16 == idx).astype(jnp.float32)
        og = jnp.sum(onehot * offs, axis=1, keepdims=True)
        pos_ref[...] = (rank_s[rows, :] + og).astype(jnp.int32)
        ss_ref[...] = jnp.broadcast_to(sc_s[rows, :], (_BTA, 128))


def _gate_positions(x, wg_red, wg, n_experts):
    t, d = x.shape
    n_tiles = t // _BTA
    tile_map = lambda p, i: (jnp.where(p == 0, n_tiles, i), 0)
    pos_full, ss_full, meta = pl.pallas_call(
        functools.partial(_gate_body, n_experts=n_experts, n_tiles=n_tiles),
        grid=(2, n_tiles),
        in_specs=[
            pl.BlockSpec((_BTA, d), lambda p, i: (jnp.where(p == 0, i, 0), 0)),
            pl.BlockSpec(wg_red.shape, lambda p, i: (0, 0)),
            pl.BlockSpec(wg.shape, lambda p, i: (0, 0)),
        ],
        out_specs=[
            pl.BlockSpec((_BTA, 1), tile_map),
            pl.BlockSpec((_BTA, 128), tile_map),
            pl.BlockSpec((4, 16), lambda p, i: (0, 0)),
        ],
        out_shape=[
            jax.ShapeDtypeStruct((t + _BTA, 1), jnp.int32),
            jax.ShapeDtypeStruct((t + _BTA, 128), jnp.float32),
            jax.ShapeDtypeStruct((4, 16), jnp.int32),
        ],
        scratch_shapes=[
            pltpu.VMEM((t, 1), jnp.int32),
            pltpu.VMEM((t, 1), jnp.float32),
            pltpu.VMEM((t, 1), jnp.float32),
            pltpu.VMEM((1, 16), jnp.float32),
        ],
        compiler_params=pltpu.CompilerParams(
            dimension_semantics=("arbitrary", "arbitrary"),
        ),
    )(x, wg_red, wg)
    return pos_full, ss_full, meta


def _sc_scatter(x, ss16, pos, t):
    d = x.shape[1]
    mesh = plsc.VectorSubcoreMesh(core_axis_name="c", subcore_axis_name="s")
    nw = 32
    b = t // nw

    @functools.partial(
        pl.kernel, mesh=mesh,
        out_type=[jax.ShapeDtypeStruct((t, d), jnp.float32),
                  jax.ShapeDtypeStruct((t, 128), jnp.float32)],
        scratch_types=[
            pltpu.VMEM((b,), jnp.int32),
            pltpu.VMEM((b, d), jnp.float32),
            pltpu.VMEM((b, 128), jnp.float32),
            pltpu.SemaphoreType.DMA,
        ],
    )
    def k(x_hbm, ss_hbm, pos_hbm, xs_hbm, sss_hbm, idx_v, rows_v, sc_v, sem):
        wid = lax.axis_index("s") * 2 + lax.axis_index("c")
        base = wid * b
        pltpu.sync_copy(pos_hbm.at[pl.ds(base, b)], idx_v)
        pltpu.sync_copy(x_hbm.at[pl.ds(base, b)], rows_v)
        pltpu.sync_copy(ss_hbm.at[pl.ds(base, b)], sc_v)
        pltpu.async_copy(rows_v, xs_hbm.at[idx_v], sem).wait()
        pltpu.async_copy(sc_v, sss_hbm.at[idx_v], sem).wait()

    return k(x, ss16, pos)


def _ffn_body(meta_ref, xs_ref, ss_ref, w1_ref, b1_ref, w2_ref,
              b2_ref, o_ref, *, n_experts):
    c = pl.program_id(0)
    s = pl.program_id(1)
    bj = meta_ref[16 + s]
    lo = meta_ref[32 + s]
    hi = meta_ref[48 + s]
    rows = pl.ds(bj * _BT, _BT)

    @pl.when(jnp.logical_and(c == 0, s == 0))
    def _zero():
        o_ref[...] = jnp.zeros_like(o_ref[...])

    @pl.when(hi > lo)
    def _compute():
        xb = xs_ref[rows, :].astype(jnp.bfloat16)
        w1 = w1_ref[0].astype(jnp.bfloat16)
        h = jnp.dot(xb, w1, preferred_element_type=jnp.float32)
        h = _gelu_exact(h + b1_ref[0])
        part = jnp.dot(h.astype(jnp.bfloat16), w2_ref[0].astype(jnp.bfloat16),
                       preferred_element_type=jnp.float32)
        part = part + jnp.where(c == 0, 1.0, 0.0) * b2_ref[0]
        y = part * ss_ref[rows, :1]
        r_io = lax.broadcasted_iota(jnp.int32, (_BT, 1), 0)
        mask = jnp.logical_and(r_io >= lo, r_io < hi)
        o_ref[rows, :] += jnp.where(mask, y, 0.0)


def _grouped_ffn(xs, sss, meta, weight1, bias1, weight2, bias2):
    t, d = xs.shape
    n_experts, _, hdim = weight1.shape
    hc = hdim // _NHC

    grid_spec = pltpu.PrefetchScalarGridSpec(
        num_scalar_prefetch=1,
        grid=(_NHC, 16),
        in_specs=[
            pl.BlockSpec((t, d), lambda c, s, m: (0, 0)),
            pl.BlockSpec((t, 128), lambda c, s, m: (0, 0)),
            pl.BlockSpec((1, d, hc), lambda c, s, m: (m[s], 0, c)),
            pl.BlockSpec((1, 1, hc), lambda c, s, m: (m[s], 0, c)),
            pl.BlockSpec((1, hc, d), lambda c, s, m: (m[s], c, 0)),
            pl.BlockSpec((1, 1, d), lambda c, s, m: (m[s], 0, 0)),
        ],
        out_specs=pl.BlockSpec((t, d), lambda c, s, m: (0, 0)),
    )
    body = functools.partial(_ffn_body, n_experts=n_experts)
    return pl.pallas_call(
        body,
        grid_spec=grid_spec,
        out_shape=jax.ShapeDtypeStruct((t, d), jnp.float32),
        compiler_params=pltpu.CompilerParams(
            dimension_semantics=("arbitrary", "arbitrary"),
        ),
    )(meta, xs, sss, weight1, bias1.reshape(n_experts, 1, hdim),
      weight2, bias2.reshape(n_experts, 1, d))


def _sc_gather(ys, pos):
    t, d = ys.shape
    mesh = plsc.VectorSubcoreMesh(core_axis_name="c", subcore_axis_name="s")
    nw = 32
    b = t // nw

    @functools.partial(
        pl.kernel, mesh=mesh,
        out_type=jax.ShapeDtypeStruct((t, d), jnp.float32),
        scratch_types=[
            pltpu.VMEM((b,), jnp.int32),
            pltpu.VMEM((b, d), jnp.float32),
            pltpu.SemaphoreType.DMA,
        ],
    )
    def k(ys_hbm, pos_hbm, y_hbm, idx_v, rows_v, sem):
        wid = lax.axis_index("s") * 2 + lax.axis_index("c")
        base = wid * b
        pltpu.sync_copy(pos_hbm.at[pl.ds(base, b)], idx_v)
        pltpu.async_copy(ys_hbm.at[idx_v], rows_v, sem).wait()
        pltpu.sync_copy(rows_v, y_hbm.at[pl.ds(base, b)])

    return k(ys, pos)


def kernel(hidden_states, wg_red, wg, weight1, bias1, weight2, bias2):
    bsz, t, d = hidden_states.shape
    n_experts = weight1.shape[0]
    x = hidden_states.reshape(t, d)

    pos, ss16, meta = _gate_positions(x, wg_red, wg, n_experts)
    pos1 = pos.reshape(-1)
    xs, sss = _sc_scatter(x, ss16, pos1, t)
    ys = _grouped_ffn(xs, sss, meta.reshape(64),
                      weight1, bias1, weight2, bias2)
    y = _sc_gather(ys, pos1)
    return y.reshape(bsz, t, d)
